# baseline (speedup 1.0000x reference)
import functools

import jax
import jax.numpy as jnp
from jax import lax
from jax.experimental import pallas as pl
from jax.experimental.pallas import tpu as pltpu
from jax.experimental.pallas import tpu_sc as plsc

_BATCH = 16384
_D = 64
_NC = 2
_NS = 16
_NW = _NC * _NS
_BPW = _BATCH // _NW
_CHUNK = 128
_NCH = _BPW // _CHUNK


def _mf_body(uid_hbm, iid_hbm, ut_hbm, it_hbm, ub_hbm, ib_hbm, gb_hbm,
             out_hbm,
             uid_v, iid_v, u_t, i_t, ub_v, ib_v, gb_v, out_v, sem):
  wid = lax.axis_index("s") * _NC + lax.axis_index("c")
  base = wid * _BPW

  for c in range(_NCH):
    pltpu.sync_copy(uid_hbm.at[pl.ds(base + c * _CHUNK, _CHUNK)], uid_v.at[c])
    pltpu.sync_copy(iid_hbm.at[pl.ds(base + c * _CHUNK, _CHUNK)], iid_v.at[c])
  pltpu.sync_copy(gb_hbm, gb_v)

  copies = []
  for j in range(_D):
    for c in range(_NCH):
      copies.append(
          pltpu.async_copy(ut_hbm.at[j].at[uid_v.at[c]], u_t.at[j, c], sem))
      copies.append(
          pltpu.async_copy(it_hbm.at[j].at[iid_v.at[c]], i_t.at[j, c], sem))
  for c in range(_NCH):
    copies.append(pltpu.async_copy(ub_hbm.at[uid_v.at[c]], ub_v.at[c], sem))
    copies.append(pltpu.async_copy(ib_hbm.at[iid_v.at[c]], ib_v.at[c], sem))
  for cp in copies:
    cp.wait()

  gbvec = gb_v[...]

  def chunk(c, carry):
    for s in range(_CHUNK // 16):
      sl = pl.ds(s * 16, 16)
      acc = u_t[0, c, sl] * i_t[0, c, sl]
      for j in range(1, _D):
        acc = acc + u_t[j, c, sl] * i_t[j, c, sl]
      out_v[c, sl] = acc + ub_v[c, sl] + ib_v[c, sl] + gbvec
    return carry

  lax.fori_loop(0, _NCH, chunk, 0)

  for c in range(_NCH):
    pltpu.sync_copy(out_v.at[c], out_hbm.at[pl.ds(base + c * _CHUNK, _CHUNK)])


@jax.jit
def _mf(uid, iid, ut, it, ub, ib, gb):
  mesh = plsc.VectorSubcoreMesh(core_axis_name="c", subcore_axis_name="s")
  f = functools.partial(
      pl.kernel,
      out_type=jax.ShapeDtypeStruct((_BATCH,), jnp.float32),
      mesh=mesh,
      compiler_params=pltpu.CompilerParams(
          needs_layout_passes=False, use_tc_tiling_on_sc=False),
      scratch_types=[
          pltpu.VMEM((_NCH, _CHUNK), jnp.int32),
          pltpu.VMEM((_NCH, _CHUNK), jnp.int32),
          pltpu.VMEM((_D, _NCH, _CHUNK), jnp.float32),
          pltpu.VMEM((_D, _NCH, _CHUNK), jnp.float32),
          pltpu.VMEM((_NCH, _CHUNK), jnp.float32),
          pltpu.VMEM((_NCH, _CHUNK), jnp.float32),
          pltpu.VMEM((16,), jnp.float32),
          pltpu.VMEM((_NCH, _CHUNK), jnp.float32),
          pltpu.SemaphoreType.DMA,
      ],
  )(_mf_body)
  return f(uid, iid, ut, it, ub, ib, gb)


def kernel(user_ids, item_ids, user_embeddings, item_embeddings, user_bias,
           item_bias, global_bias):
  uid = user_ids.astype(jnp.int32)
  iid = item_ids.astype(jnp.int32)
  ut = user_embeddings.T
  it = item_embeddings.T
  ub = user_bias.reshape(-1)
  ib = item_bias.reshape(-1)
  gb16 = jnp.broadcast_to(global_bias.reshape(-1), (16,))
  return _mf(uid, iid, ut, it, ub, ib, gb16)
